# baseline (speedup 1.0000x reference)
import dataclasses
import functools

import jax
import jax.numpy as jnp
from jax import lax
from jax.experimental import pallas as pl
from jax.experimental.pallas import tpu as pltpu
from jax.experimental.pallas import tpu_sc as plsc

N = 524288
D = 8
NUM_WORKERS = 32
Q_PER_WORKER = N // NUM_WORKERS
SB = 512
NSB = Q_PER_WORKER // SB
NJ = SB // 128
CHUNK = 8
NUM_CHUNKS = N // CHUNK
LANES = 16


FRONT_C = 4096
FRONT_G = N // FRONT_C


def _front_body(st_ref, ep_ref, on_ref, lw_ref, t_ref, s_ref, sc_ref, sg_ref,
                pm_ref, cp_ref, mc_ref, fg_ref, ns_ref, lo_ref):
    st = st_ref[...]
    ep = ep_ref[...]
    ns = st + (ep * sg_ref[...]) * sc_ref[...]
    d = ns - pm_ref[...]
    sq = cp_ref[...] * (d * d)
    x = ns[0:1]
    sp = jnp.where(x != x, x, jnp.maximum(x, 0.0) + jnp.log1p(jnp.exp(-jnp.abs(x))))
    neg_rate = -sp
    dt = jnp.maximum(t_ref[...] - on_ref[...], 0.0)
    p = jnp.exp(neg_rate * dt)
    diff = s_ref[...] - p
    sq2 = diff * diff
    s_pr = jnp.sum(sq, axis=0, keepdims=True)
    s_ll = jnp.sum(sq2, axis=0, keepdims=True)
    logw = (fg_ref[0, 0] * lw_ref[...] + s_ll * mc_ref[0, 0]) + s_pr * (-0.5)
    ns_ref[...] = ns
    lo_ref[...] = logw


def _front_tc(states_T, eps_T, on2, lw2, t32, s32, scale, sigma, pm, cp, m05c, fgt):
    blk8 = lambda i: (0, i)
    small = pl.BlockSpec((8, 1), lambda i: (0, 0))
    one = pl.BlockSpec((1, 1), lambda i: (0, 0))
    return pl.pallas_call(
        _front_body,
        grid=(FRONT_G,),
        in_specs=[
            pl.BlockSpec((8, FRONT_C), blk8),
            pl.BlockSpec((8, FRONT_C), blk8),
            pl.BlockSpec((1, FRONT_C), blk8),
            pl.BlockSpec((1, FRONT_C), blk8),
            pl.BlockSpec((32, 1), lambda i: (0, 0)),
            pl.BlockSpec((32, 1), lambda i: (0, 0)),
            small, small, small, small, one, one,
        ],
        out_specs=[
            pl.BlockSpec((8, FRONT_C), blk8),
            pl.BlockSpec((1, FRONT_C), blk8),
        ],
        out_shape=[
            jax.ShapeDtypeStruct((8, N), jnp.float32),
            jax.ShapeDtypeStruct((1, N), jnp.float32),
        ],
    )(states_T, eps_T, on2, lw2, t32, s32, scale, sigma, pm, cp, m05c, fgt)


def _sc_compiler_params():
    cp = pltpu.CompilerParams()
    fields = pltpu.CompilerParams.__dataclass_fields__
    if "needs_layout_passes" in fields:
        cp = dataclasses.replace(cp, needs_layout_passes=False)
    if "use_tc_tiling_on_sc" in fields:
        cp = dataclasses.replace(cp, use_tc_tiling_on_sc=False)
    return cp


def _resample_sc(cdf8, chunk_cdf, u_samples, new_states):
    mesh = plsc.VectorSubcoreMesh(core_axis_name="c", subcore_axis_name="s")

    @functools.partial(
        pl.kernel,
        out_type=jax.ShapeDtypeStruct((N, D), jnp.float32),
        mesh=mesh,
        compiler_params=_sc_compiler_params(),
        scratch_types=[
            pltpu.VMEM((NUM_CHUNKS,), jnp.float32),
            pltpu.VMEM((2, SB), jnp.float32),
            pltpu.VMEM((NJ, 128), jnp.int32),
            pltpu.VMEM((SB, CHUNK), jnp.float32),
            pltpu.VMEM((NJ, 128), jnp.int32),
            pltpu.VMEM((2, SB, D), jnp.float32),
            pltpu.SemaphoreType.DMA,
            pltpu.SemaphoreType.DMA,
            pltpu.SemaphoreType.DMA,
            pltpu.SemaphoreType.DMA,
            pltpu.SemaphoreType.DMA,
            pltpu.SemaphoreType.DMA,
        ],
    )
    def k(cdf8_hbm, t_hbm, u_hbm, ns_hbm, out_hbm,
          t_v, u_v, c_v, rows_v, i_v, o_v, su0, su1, sg, sn, so0, so1):
        wid = lax.axis_index("s") * 2 + lax.axis_index("c")
        base_q = wid * Q_PER_WORKER
        su = (su0, su1)
        so = (so0, so1)
        pltpu.sync_copy(t_hbm, t_v)
        pltpu.async_copy(u_hbm.at[pl.ds(base_q, SB)], u_v.at[0], su0)

        @pl.loop(0, NSB // 2)
        def _(g):
            for par in (0, 1):
                s = g * 2 + par
                qb = base_q + s * SB

                @pl.when(g >= 1)
                def _():
                    pltpu.make_async_copy(
                        o_v.at[par], out_hbm.at[pl.ds(base_q, SB)], so[par]
                    ).wait()

                pltpu.make_async_copy(
                    u_hbm.at[pl.ds(base_q, SB)], u_v.at[par], su[par]
                ).wait()

                @pl.when(s + 1 < NSB)
                def _():
                    pltpu.async_copy(
                        u_hbm.at[pl.ds(qb + SB, SB)], u_v.at[1 - par], su[1 - par]
                    )

                g_handles = []
                for j in range(NJ):
                    @pl.loop(0, 128, step=LANES)
                    def _(voff, j=j):
                        u16 = u_v[par, pl.ds(j * 128 + voff, LANES)]
                        pos = jnp.zeros((LANES,), jnp.int32)
                        for bit in [1 << b for b in range(15, -1, -1)]:
                            probe = pos + (bit - 1)
                            val = plsc.load_gather(t_v, [probe])
                            pos = pos + jnp.where(val < u16, jnp.int32(bit), jnp.int32(0))
                        c_v[j, pl.ds(voff, LANES)] = pos

                    g_handles.append(
                        pltpu.async_copy(
                            cdf8_hbm.at[c_v.at[j]],
                            rows_v.at[pl.ds(j * 128, 128)],
                            sg,
                        )
                    )

                n_handles = []
                for j in range(NJ):
                    g_handles[j].wait()

                    @pl.loop(0, 128, step=LANES)
                    def _(voff, j=j):
                        u16 = u_v[par, pl.ds(j * 128 + voff, LANES)]
                        c16 = c_v[j, pl.ds(voff, LANES)]
                        row = j * 128 + voff + lax.iota(jnp.int32, LANES)
                        cnt = jnp.zeros((LANES,), jnp.int32)
                        for kk in range(CHUNK - 1):
                            col = jnp.full((LANES,), kk, jnp.int32)
                            vals = plsc.load_gather(rows_v, [row, col])
                            cnt = cnt + jnp.where(vals < u16, jnp.int32(1), jnp.int32(0))
                        i_v[j, pl.ds(voff, LANES)] = jnp.minimum(
                            c16 * CHUNK + cnt, jnp.int32(N - 1)
                        )

                    n_handles.append(
                        pltpu.async_copy(
                            ns_hbm.at[i_v.at[j]],
                            o_v.at[par, pl.ds(j * 128, 128)],
                            sn,
                        )
                    )

                for j in range(NJ):
                    n_handles[j].wait()
                pltpu.async_copy(o_v.at[par], out_hbm.at[pl.ds(qb, SB)], so[par])

        pltpu.make_async_copy(o_v.at[0], out_hbm.at[pl.ds(base_q, SB)], so0).wait()
        pltpu.make_async_copy(o_v.at[1], out_hbm.at[pl.ds(base_q, SB)], so1).wait()

    return k(cdf8, chunk_cdf, u_samples, new_states)


def kernel(t_obs, s_obs, states, log_weights, onsets, sigma, noise_eps, u_samples, W1, b1, W2, b2, W3, b3):
    d = states.shape[1]
    Bn = t_obs.shape[0]
    mean_loglik = jnp.zeros((Bn,), dtype=t_obs.dtype)
    std_loglik = jnp.zeros((Bn,), dtype=t_obs.dtype)
    ess = jnp.ones((Bn,), dtype=t_obs.dtype)
    x = jnp.stack([t_obs / 100.0, s_obs, jnp.tanh(mean_loglik / 50.0), jnp.tanh(std_loglik / 10.0), ess], axis=-1)
    h = jax.nn.relu(x @ W1 + b1)
    h = jax.nn.relu(h @ W2 + b2)
    out = jax.nn.softplus(h @ W3 + b3)
    out_mean = out.mean(axis=0)
    noise_scale = out_mean[:d]
    correction = out_mean[d:]
    correct_prior = correction[:d]
    correct_lik = correction[-2]
    forget_lik = correction[-1]
    prior_mean = jnp.mean(states, axis=0)
    m05c = correct_lik * jnp.float32(-0.5)
    ns_t, logw2 = _front_tc(
        states.T, noise_eps.T, onsets.reshape(1, N), log_weights.reshape(1, N),
        t_obs.reshape(32, 1), s_obs.reshape(32, 1), noise_scale.reshape(8, 1),
        sigma.reshape(8, 1), prior_mean.reshape(8, 1), correct_prior.reshape(8, 1),
        m05c.reshape(1, 1), forget_lik.reshape(1, 1))
    new_states = ns_t.T
    new_logw = logw2.reshape(N)
    weights = jax.nn.softmax(new_logw)
    cdf = jnp.cumsum(weights)
    cdf8 = cdf.reshape(NUM_CHUNKS, CHUNK)
    chunk_cdf = cdf8[:, CHUNK - 1]
    return _resample_sc(cdf8, chunk_cdf, u_samples, new_states)


# speedup vs baseline: 1.0285x; 1.0285x over previous
import dataclasses
import functools

import jax
import jax.numpy as jnp
from jax import lax
from jax.experimental import pallas as pl
from jax.experimental.pallas import tpu as pltpu
from jax.experimental.pallas import tpu_sc as plsc

N = 524288
D = 8
NUM_WORKERS = 32
Q_PER_WORKER = N // NUM_WORKERS
SB = 512
NSB = Q_PER_WORKER // SB
NJ = SB // 128
CHUNK = 8
NUM_CHUNKS = N // CHUNK
LANES = 16


FRONT_C = 16384
FRONT_G = N // FRONT_C


def _front_body(st_ref, ep_ref, on_ref, lw_ref, t_ref, s_ref, sc_ref, sg_ref,
                pm_ref, cp_ref, mc_ref, fg_ref, ns_ref, lo_ref):
    st = st_ref[...]
    ep = ep_ref[...]
    ns = st + (ep * sg_ref[...]) * sc_ref[...]
    d = ns - pm_ref[...]
    sq = cp_ref[...] * (d * d)
    x = ns[0:1]
    sp = jnp.where(x != x, x, jnp.maximum(x, 0.0) + jnp.log1p(jnp.exp(-jnp.abs(x))))
    neg_rate = -sp
    dt = jnp.maximum(t_ref[...] - on_ref[...], 0.0)
    p = jnp.exp(neg_rate * dt)
    diff = s_ref[...] - p
    sq2 = diff * diff
    s_pr = jnp.sum(sq, axis=0, keepdims=True)
    s_ll = jnp.sum(sq2, axis=0, keepdims=True)
    logw = (fg_ref[0, 0] * lw_ref[...] + s_ll * mc_ref[0, 0]) + s_pr * (-0.5)
    ns_ref[...] = ns
    lo_ref[...] = logw


def _front_tc(states_T, eps_T, on2, lw2, t32, s32, scale, sigma, pm, cp, m05c, fgt):
    blk8 = lambda i: (0, i)
    small = pl.BlockSpec((8, 1), lambda i: (0, 0))
    one = pl.BlockSpec((1, 1), lambda i: (0, 0))
    return pl.pallas_call(
        _front_body,
        grid=(FRONT_G,),
        in_specs=[
            pl.BlockSpec((8, FRONT_C), blk8),
            pl.BlockSpec((8, FRONT_C), blk8),
            pl.BlockSpec((1, FRONT_C), blk8),
            pl.BlockSpec((1, FRONT_C), blk8),
            pl.BlockSpec((32, 1), lambda i: (0, 0)),
            pl.BlockSpec((32, 1), lambda i: (0, 0)),
            small, small, small, small, one, one,
        ],
        out_specs=[
            pl.BlockSpec((8, FRONT_C), blk8),
            pl.BlockSpec((1, FRONT_C), blk8),
        ],
        out_shape=[
            jax.ShapeDtypeStruct((8, N), jnp.float32),
            jax.ShapeDtypeStruct((1, N), jnp.float32),
        ],
    )(states_T, eps_T, on2, lw2, t32, s32, scale, sigma, pm, cp, m05c, fgt)


def _sc_compiler_params():
    cp = pltpu.CompilerParams()
    fields = pltpu.CompilerParams.__dataclass_fields__
    if "needs_layout_passes" in fields:
        cp = dataclasses.replace(cp, needs_layout_passes=False)
    if "use_tc_tiling_on_sc" in fields:
        cp = dataclasses.replace(cp, use_tc_tiling_on_sc=False)
    return cp


def _resample_sc(cdf8, chunk_cdf, u_samples, new_states):
    mesh = plsc.VectorSubcoreMesh(core_axis_name="c", subcore_axis_name="s")

    @functools.partial(
        pl.kernel,
        out_type=jax.ShapeDtypeStruct((N, D), jnp.float32),
        mesh=mesh,
        compiler_params=_sc_compiler_params(),
        scratch_types=[
            pltpu.VMEM((NUM_CHUNKS,), jnp.float32),
            pltpu.VMEM((2, SB), jnp.float32),
            pltpu.VMEM((NJ, 128), jnp.int32),
            pltpu.VMEM((SB, CHUNK), jnp.float32),
            pltpu.VMEM((NJ, 128), jnp.int32),
            pltpu.VMEM((2, SB, D), jnp.float32),
            pltpu.SemaphoreType.DMA,
            pltpu.SemaphoreType.DMA,
            pltpu.SemaphoreType.DMA,
            pltpu.SemaphoreType.DMA,
            pltpu.SemaphoreType.DMA,
            pltpu.SemaphoreType.DMA,
        ],
    )
    def k(cdf8_hbm, t_hbm, u_hbm, ns_hbm, out_hbm,
          t_v, u_v, c_v, rows_v, i_v, o_v, su0, su1, sg, sn, so0, so1):
        wid = lax.axis_index("s") * 2 + lax.axis_index("c")
        base_q = wid * Q_PER_WORKER
        su = (su0, su1)
        so = (so0, so1)
        pltpu.sync_copy(t_hbm, t_v)
        pltpu.async_copy(u_hbm.at[pl.ds(base_q, SB)], u_v.at[0], su0)

        @pl.loop(0, NSB // 2)
        def _(g):
            for par in (0, 1):
                s = g * 2 + par
                qb = base_q + s * SB

                @pl.when(g >= 1)
                def _():
                    pltpu.make_async_copy(
                        o_v.at[par], out_hbm.at[pl.ds(base_q, SB)], so[par]
                    ).wait()

                pltpu.make_async_copy(
                    u_hbm.at[pl.ds(base_q, SB)], u_v.at[par], su[par]
                ).wait()

                @pl.when(s + 1 < NSB)
                def _():
                    pltpu.async_copy(
                        u_hbm.at[pl.ds(qb + SB, SB)], u_v.at[1 - par], su[1 - par]
                    )

                g_handles = []
                for j in range(NJ):
                    @pl.loop(0, 128, step=LANES)
                    def _(voff, j=j):
                        u16 = u_v[par, pl.ds(j * 128 + voff, LANES)]
                        pos = jnp.zeros((LANES,), jnp.int32)
                        for bit in [1 << b for b in range(15, -1, -1)]:
                            probe = pos + (bit - 1)
                            val = plsc.load_gather(t_v, [probe])
                            pos = pos + jnp.where(val < u16, jnp.int32(bit), jnp.int32(0))
                        c_v[j, pl.ds(voff, LANES)] = pos

                    g_handles.append(
                        pltpu.async_copy(
                            cdf8_hbm.at[c_v.at[j]],
                            rows_v.at[pl.ds(j * 128, 128)],
                            sg,
                        )
                    )

                n_handles = []
                for j in range(NJ):
                    g_handles[j].wait()

                    @pl.loop(0, 128, step=LANES)
                    def _(voff, j=j):
                        u16 = u_v[par, pl.ds(j * 128 + voff, LANES)]
                        c16 = c_v[j, pl.ds(voff, LANES)]
                        row = j * 128 + voff + lax.iota(jnp.int32, LANES)
                        cnt = jnp.zeros((LANES,), jnp.int32)
                        for kk in range(CHUNK - 1):
                            col = jnp.full((LANES,), kk, jnp.int32)
                            vals = plsc.load_gather(rows_v, [row, col])
                            cnt = cnt + jnp.where(vals < u16, jnp.int32(1), jnp.int32(0))
                        i_v[j, pl.ds(voff, LANES)] = jnp.minimum(
                            c16 * CHUNK + cnt, jnp.int32(N - 1)
                        )

                    n_handles.append(
                        pltpu.async_copy(
                            ns_hbm.at[i_v.at[j]],
                            o_v.at[par, pl.ds(j * 128, 128)],
                            sn,
                        )
                    )

                for j in range(NJ):
                    n_handles[j].wait()
                pltpu.async_copy(o_v.at[par], out_hbm.at[pl.ds(qb, SB)], so[par])

        pltpu.make_async_copy(o_v.at[0], out_hbm.at[pl.ds(base_q, SB)], so0).wait()
        pltpu.make_async_copy(o_v.at[1], out_hbm.at[pl.ds(base_q, SB)], so1).wait()

    return k(cdf8, chunk_cdf, u_samples, new_states)


def kernel(t_obs, s_obs, states, log_weights, onsets, sigma, noise_eps, u_samples, W1, b1, W2, b2, W3, b3):
    d = states.shape[1]
    Bn = t_obs.shape[0]
    mean_loglik = jnp.zeros((Bn,), dtype=t_obs.dtype)
    std_loglik = jnp.zeros((Bn,), dtype=t_obs.dtype)
    ess = jnp.ones((Bn,), dtype=t_obs.dtype)
    x = jnp.stack([t_obs / 100.0, s_obs, jnp.tanh(mean_loglik / 50.0), jnp.tanh(std_loglik / 10.0), ess], axis=-1)
    h = jax.nn.relu(x @ W1 + b1)
    h = jax.nn.relu(h @ W2 + b2)
    out = jax.nn.softplus(h @ W3 + b3)
    out_mean = out.mean(axis=0)
    noise_scale = out_mean[:d]
    correction = out_mean[d:]
    correct_prior = correction[:d]
    correct_lik = correction[-2]
    forget_lik = correction[-1]
    prior_mean = jnp.mean(states, axis=0)
    m05c = correct_lik * jnp.float32(-0.5)
    ns_t, logw2 = _front_tc(
        states.T, noise_eps.T, onsets.reshape(1, N), log_weights.reshape(1, N),
        t_obs.reshape(32, 1), s_obs.reshape(32, 1), noise_scale.reshape(8, 1),
        sigma.reshape(8, 1), prior_mean.reshape(8, 1), correct_prior.reshape(8, 1),
        m05c.reshape(1, 1), forget_lik.reshape(1, 1))
    new_states = ns_t.T
    new_logw = logw2.reshape(N)
    weights = jax.nn.softmax(new_logw)
    cdf = jnp.cumsum(weights)
    cdf8 = cdf.reshape(NUM_CHUNKS, CHUNK)
    chunk_cdf = cdf8[:, CHUNK - 1]
    return _resample_sc(cdf8, chunk_cdf, u_samples, new_states)
